# baseline (speedup 1.0000x reference)
import functools

import numpy as np

import jax
import jax.numpy as jnp
from jax import lax
from jax.experimental import pallas as pl
from jax.experimental.pallas import tpu as pltpu
from jax.experimental.pallas import tpu_sc as plsc

D = 128
BATCH = 1024
S1 = 10
S2 = 25
N1 = BATCH * S1
LANES = 16
NV = D // LANES

NC = 2
NS = 16
NW = NC * NS

H0_PW = BATCH // NW
H1_PW = N1 // NW
H1_CHUNK = 80
N1_CHUNKS = H1_PW // H1_CHUNK
SEG_PW = N1 // NW
CH_SEGS = 8
CH_ROWS = CH_SEGS * S2
CH_SPLIT = 104
N_CH = SEG_PW // CH_SEGS
NBUF = 2
N_ROUNDS = N_CH // NBUF

_f32 = jnp.float32


def _seg_mean(rows, base, n, scale, out, out_row):
    unroll = 5
    def body(r5, acc):
        for u in range(unroll):
            r = base + r5 * unroll + u
            acc = tuple(acc[j] + rows[r, pl.ds(LANES * j, LANES)]
                        for j in range(NV))
        return acc
    acc = lax.fori_loop(0, n // unroll, body,
                        tuple(jnp.zeros((LANES,), _f32) for _ in range(NV)))
    for j in range(NV):
        out[out_row, pl.ds(LANES * j, LANES)] = acc[j] * scale


@functools.partial(
    pl.kernel,
    mesh=plsc.VectorSubcoreMesh(core_axis_name="c", subcore_axis_name="s"),
    out_type=(
        jax.ShapeDtypeStruct((BATCH, D), _f32),
        jax.ShapeDtypeStruct((BATCH, D), _f32),
        jax.ShapeDtypeStruct((N1, D), _f32),
        jax.ShapeDtypeStruct((N1, D), _f32),
    ),
    scratch_types=(
        pltpu.VMEM((H0_PW,), jnp.int32),
        pltpu.VMEM((H1_PW,), jnp.int32),
        pltpu.VMEM((SEG_PW * S2,), jnp.int32),
        pltpu.VMEM((H0_PW, D), _f32),
        pltpu.VMEM((H1_CHUNK, D), _f32),
        pltpu.VMEM((H0_PW, D), _f32),
        pltpu.VMEM((NBUF, CH_ROWS, D), _f32),
        pltpu.VMEM((NBUF, CH_SEGS, D), _f32),
        pltpu.SemaphoreType.DMA,
        pltpu.SemaphoreType.DMA,
        pltpu.SemaphoreType.DMA,
        pltpu.SemaphoreType.DMA,
        pltpu.SemaphoreType.DMA,
    ),
)
def _sc_gather(feat, s0, s1, s2, h0_out, m1_out, h1_out, m2_out,
               idx0, idx1, idx2, rows0, rows1, m1b, rows2, m2st,
               sem, semg0, semg1, semw0, semw1):
    semg = (semg0, semg1)
    semw = (semw0, semw1)
    wid = lax.axis_index("s") * NC + lax.axis_index("c")

    def _m2_gathers(g, b):
        base = g * CH_ROWS
        lo = pltpu.make_async_copy(
            feat.at[idx2.at[pl.ds(base, CH_SPLIT)]],
            rows2.at[b].at[pl.ds(0, CH_SPLIT)], semg[b])
        hi = pltpu.make_async_copy(
            feat.at[idx2.at[pl.ds(base + CH_SPLIT, CH_ROWS - CH_SPLIT)]],
            rows2.at[b].at[pl.ds(CH_SPLIT, CH_ROWS - CH_SPLIT)], semg[b])
        return lo, hi

    pltpu.sync_copy(s2.at[pl.ds(wid * (SEG_PW * S2), SEG_PW * S2)], idx2)
    for b in range(NBUF):
        for cp in _m2_gathers(b, b):
            cp.start()

    pltpu.sync_copy(s0.at[pl.ds(wid * H0_PW, H0_PW)], idx0)
    pltpu.async_copy(feat.at[idx0], rows0, sem).wait()
    pltpu.sync_copy(rows0, h0_out.at[pl.ds(wid * H0_PW, H0_PW)])

    pltpu.sync_copy(s1.at[pl.ds(wid * H1_PW, H1_PW)], idx1)
    for c in range(N1_CHUNKS):
        pltpu.async_copy(
            feat.at[idx1.at[pl.ds(c * H1_CHUNK, H1_CHUNK)]], rows1,
            sem).wait()
        pltpu.sync_copy(
            rows1, h1_out.at[pl.ds(wid * H1_PW + c * H1_CHUNK, H1_CHUNK)])

        def m1_seg(s, carry):
            _seg_mean(rows1, S1 * s, S1, 1.0 / S1, m1b,
                      c * (H1_CHUNK // S1) + s)
            return carry

        lax.fori_loop(0, H1_CHUNK // S1, m1_seg, 0)
    pltpu.sync_copy(m1b, m1_out.at[pl.ds(wid * H0_PW, H0_PW)])

    def _m2_write(g, b):
        return pltpu.make_async_copy(
            m2st.at[b],
            m2_out.at[pl.ds(wid * SEG_PW + CH_SEGS * g, CH_SEGS)], semw[b])

    def m2_round(i, carry):
        for b in range(NBUF):
            g = NBUF * i + b
            for cp in _m2_gathers(g, b):
                cp.wait()
            buf = rows2.at[b]

            @pl.when(i > 0)
            def _():
                _m2_write(g - NBUF, b).wait()

            def m2_seg(s, carry2):
                _seg_mean(buf, S2 * s, S2, 1.0 / S2, m2st.at[b], s)
                return carry2

            lax.fori_loop(0, CH_SEGS, m2_seg, 0)
            _m2_write(g, b).start()

            @pl.when(i < N_ROUNDS - 1)
            def _():
                for cp in _m2_gathers(g + NBUF, b):
                    cp.start()
        return carry

    lax.fori_loop(0, N_ROUNDS, m2_round, 0)
    for b in range(NBUF):
        _m2_write(NBUF * (N_ROUNDS - 1) + b, b).wait()


_BLK = 1280
_NT = N1 // _BLK


def _tc_dense(h0, m1, h1, m2, ws0, wn0, ws1, wn1, p0, out):
    t = pl.program_id(0)
    dot = lambda a, b: lax.dot(a, b, preferred_element_type=_f32)
    relu = lambda x: jnp.maximum(x, 0.0)
    ws0v = ws0[:]
    wn0v = wn0[:]

    @pl.when(t == 0)
    def _():
        a = relu(dot(h0[:], ws0v))
        b = relu(dot(m1[:], wn0v))
        out[:, 0:D] = dot(a, ws1[0:D, :]) + dot(b, ws1[D:2 * D, :])

    xs = relu(dot(h1[:], ws0v))
    xn = relu(dot(m2[:], wn0v))
    ms = dot(p0[:], xs)
    mn = dot(p0[:], xn)
    res = dot(ms, wn1[0:D, :]) + dot(mn, wn1[D:2 * D, :])
    out[pl.ds(t * (_BLK // S1), _BLK // S1), D:2 * D] = res


_tc_call = pl.pallas_call(
    _tc_dense,
    grid=(_NT,),
    in_specs=[
        pl.BlockSpec((BATCH, D), lambda t: (0, 0)),
        pl.BlockSpec((BATCH, D), lambda t: (0, 0)),
        pl.BlockSpec((_BLK, D), lambda t: (t, 0)),
        pl.BlockSpec((_BLK, D), lambda t: (t, 0)),
        pl.BlockSpec((D, D), lambda t: (0, 0)),
        pl.BlockSpec((D, D), lambda t: (0, 0)),
        pl.BlockSpec((2 * D, D), lambda t: (0, 0)),
        pl.BlockSpec((2 * D, D), lambda t: (0, 0)),
        pl.BlockSpec((_BLK // S1, _BLK), lambda t: (0, 0)),
    ],
    out_specs=pl.BlockSpec((BATCH, 2 * D), lambda t: (0, 0)),
    out_shape=jax.ShapeDtypeStruct((BATCH, 2 * D), _f32),
)

_P0 = np.asarray(
    (np.arange(1280)[None, :] // S1 == np.arange(128)[:, None]) / S1,
    dtype=np.float32)


def kernel(features, samples_0, samples_1, samples_2,
           W_self_0, W_neigh_0, W_self_1, W_neigh_1):
    h0, m1, h1, m2 = _sc_gather(features, samples_0, samples_1, samples_2)
    return _tc_call(h0, m1, h1, m2, W_self_0, W_neigh_0, W_self_1, W_neigh_1,
                    _P0)
